# baseline (speedup 1.0000x reference)
import functools

import jax
import jax.numpy as jnp
from jax import lax
from jax.experimental import pallas as pl
from jax.experimental.pallas import tpu as pltpu
from jax.experimental.pallas import tpu_sc as plsc

_B = 16384
_D_TAB = 56
_D_OUT = 64
_CH = 128


@functools.cache
def _make_sc_kernel():
    info = plsc.get_sparse_core_info()
    nw = info.num_cores * info.num_subcores
    b_per_w = _B // nw
    n_ch = b_per_w // _CH
    mesh = plsc.VectorSubcoreMesh(core_axis_name="c", subcore_axis_name="s")

    @functools.partial(
        pl.kernel,
        mesh=mesh,
        out_type=jax.ShapeDtypeStruct((_B, _D_OUT), jnp.float32),
        scratch_types=[
            pltpu.VMEM((b_per_w,), jnp.int32),
            pltpu.VMEM((b_per_w, _D_TAB), jnp.float32),
            pltpu.VMEM((b_per_w, _D_OUT), jnp.float32),
            pltpu.VMEM((16,), jnp.float32),
            pltpu.SemaphoreType.DMA,
        ],
        compiler_params=pltpu.CompilerParams(use_tc_tiling_on_sc=False),
    )
    def k(
        x_hbm, table_hbm, tail_hbm, out_hbm,
        idx_v, raw_v, rows_v, tail_v, sem,
    ):
        wid = lax.axis_index("s") * info.num_cores + lax.axis_index("c")
        base = wid * b_per_w

        pltpu.sync_copy(x_hbm.at[pl.ds(base, b_per_w)], idx_v)
        pltpu.sync_copy(tail_hbm, tail_v)
        v_tail = tail_v[...]

        for c in range(n_ch):
            pltpu.async_copy(
                table_hbm.at[idx_v.at[pl.ds(c * _CH, _CH)]],
                raw_v.at[pl.ds(c * _CH, _CH)],
                sem,
            )
        pltpu.make_async_copy(
            table_hbm.at[pl.ds(0, b_per_w)], raw_v, sem
        ).wait()

        def body(j, c):
            rows_v[j, pl.ds(48, 16)] = v_tail
            rows_v[j, pl.ds(0, 16)] = raw_v[j, pl.ds(0, 16)]
            rows_v[j, pl.ds(16, 16)] = raw_v[j, pl.ds(16, 16)]
            rows_v[j, pl.ds(32, 16)] = raw_v[j, pl.ds(32, 16)]
            rows_v[j, pl.ds(40, 16)] = raw_v[j, pl.ds(40, 16)]
            return c

        lax.fori_loop(0, b_per_w, body, 0)

        pltpu.sync_copy(rows_v, out_hbm.at[pl.ds(base, b_per_w)])

    return k


def kernel(x, table, shared):
    tail16 = jnp.tile(jnp.reshape(shared, (_D_OUT - _D_TAB,)), 2)
    out = _make_sc_kernel()(x.astype(jnp.int32), table, tail16)
    return out[:, None, :]


# speedup vs baseline: 1.2239x; 1.2239x over previous
import functools

import jax
import jax.numpy as jnp
from jax import lax
from jax.experimental import pallas as pl
from jax.experimental.pallas import tpu as pltpu
from jax.experimental.pallas import tpu_sc as plsc

_B = 16384
_D_TAB = 56
_D_OUT = 64
_BLK = 8
_K = 16


@functools.cache
def _make_sc_kernel():
    info = plsc.get_sparse_core_info()
    nw = info.num_cores * info.num_subcores
    b_per_w = _B // nw
    n_ch = b_per_w // _K
    mesh = plsc.VectorSubcoreMesh(core_axis_name="c", subcore_axis_name="s")

    @functools.partial(
        pl.kernel,
        mesh=mesh,
        out_type=jax.ShapeDtypeStruct((_B, _D_OUT), jnp.float32),
        scratch_types=[
            pltpu.VMEM((b_per_w,), jnp.int32),
            pltpu.VMEM((_K, _BLK, _D_TAB), jnp.float32),
            pltpu.VMEM((b_per_w, _D_OUT), jnp.float32),
            pltpu.VMEM((16,), jnp.float32),
            pltpu.VMEM_SHARED((info.num_subcores, b_per_w), jnp.int32),
            pltpu.SMEM((b_per_w,), jnp.int32),
            pltpu.SemaphoreType.DMA,
        ],
        compiler_params=pltpu.CompilerParams(use_tc_tiling_on_sc=True),
    )
    def k(
        x_hbm, table_hbm, tail_hbm, out_hbm,
        idx_v, raw_v, rows_v, tail_v, idx_sh, idx_s, sem,
    ):
        wid = lax.axis_index("s") * info.num_cores + lax.axis_index("c")
        base = wid * b_per_w

        sid = lax.axis_index("s")
        pltpu.sync_copy(x_hbm.at[pl.ds(base, b_per_w)], idx_v)
        pltpu.sync_copy(idx_v, idx_sh.at[sid])
        pltpu.sync_copy(idx_sh.at[sid], idx_s)
        pltpu.sync_copy(tail_hbm, tail_v)
        v_tail = tail_v[...]

        def chunk(c, carry):
            r0 = c * _K
            for j in range(_K):
                i = idx_s[r0 + j]
                pltpu.async_copy(
                    table_hbm.at[pl.ds(i // _BLK, 1), pl.ds(i % _BLK, 1)],
                    raw_v.at[pl.ds(j, 1), pl.ds(0, 1)],
                    sem,
                ).wait()

            def body(j, cc):
                r = r0 + j
                rows_v[r, pl.ds(48, 16)] = v_tail
                rows_v[r, pl.ds(0, 16)] = raw_v[j, 0, pl.ds(0, 16)]
                rows_v[r, pl.ds(16, 16)] = raw_v[j, 0, pl.ds(16, 16)]
                rows_v[r, pl.ds(32, 16)] = raw_v[j, 0, pl.ds(32, 16)]
                rows_v[r, pl.ds(40, 16)] = raw_v[j, 0, pl.ds(40, 16)]
                return cc

            lax.fori_loop(0, _K, body, 0)
            return carry

        lax.fori_loop(0, n_ch, chunk, 0, unroll=False)

        pltpu.sync_copy(rows_v, out_hbm.at[pl.ds(base, b_per_w)])

    return k


def kernel(x, table, shared):
    table3 = jnp.reshape(table, (table.shape[0] // _BLK, _BLK, _D_TAB))
    tail16 = jnp.tile(jnp.reshape(shared, (_D_OUT - _D_TAB,)), 2)
    out = _make_sc_kernel()(x.astype(jnp.int32), table3, tail16)
    return out[:, None, :]


# speedup vs baseline: 1.4341x; 1.1717x over previous
import functools

import jax
import jax.numpy as jnp
from jax import lax
from jax.experimental import pallas as pl
from jax.experimental.pallas import tpu as pltpu
from jax.experimental.pallas import tpu_sc as plsc

_B = 16384
_D_TAB = 56
_D_OUT = 64
_BLK = 8
_K = 16
_F = 4


@functools.cache
def _make_sc_kernel():
    info = plsc.get_sparse_core_info()
    nw = info.num_cores * info.num_subcores
    b_per_w = _B // nw
    n_ch = b_per_w // _K
    mesh = plsc.VectorSubcoreMesh(core_axis_name="c", subcore_axis_name="s")

    @functools.partial(
        pl.kernel,
        mesh=mesh,
        out_type=jax.ShapeDtypeStruct((_B, _D_OUT), jnp.float32),
        scratch_types=[
            pltpu.VMEM((b_per_w,), jnp.int32),
            pltpu.VMEM((_K, _BLK, _D_TAB), jnp.float32),
            pltpu.VMEM((b_per_w, _D_OUT), jnp.float32),
            pltpu.VMEM((16,), jnp.float32),
            pltpu.VMEM_SHARED((info.num_subcores, b_per_w), jnp.int32),
            pltpu.SMEM((b_per_w,), jnp.int32),
            [pltpu.SemaphoreType.DMA] * _F,
        ],
        compiler_params=pltpu.CompilerParams(use_tc_tiling_on_sc=True),
    )
    def k(
        x_hbm, table_hbm, tail_hbm, out_hbm,
        idx_v, raw_v, rows_v, tail_v, idx_sh, idx_s, sems,
    ):
        wid = lax.axis_index("s") * info.num_cores + lax.axis_index("c")
        base = wid * b_per_w

        sid = lax.axis_index("s")
        pltpu.sync_copy(x_hbm.at[pl.ds(base, b_per_w)], idx_v)
        pltpu.sync_copy(idx_v, idx_sh.at[sid])
        pltpu.sync_copy(idx_sh.at[sid], idx_s)
        pltpu.sync_copy(tail_hbm, tail_v)
        v_tail = tail_v[...]

        def chunk(c, carry):
            r0 = c * _K
            for g in range(_K // _F):
                copies = []
                for f in range(_F):
                    j = g * _F + f
                    i = idx_s[r0 + j]
                    copies.append(
                        pltpu.async_copy(
                            table_hbm.at[pl.ds(i // _BLK, 1), pl.ds(i % _BLK, 1)],
                            raw_v.at[pl.ds(j, 1), pl.ds(0, 1)],
                            sems[f],
                        )
                    )
                for cp in copies:
                    cp.wait()

            def body(j, cc):
                r = r0 + j
                rows_v[r, pl.ds(48, 16)] = v_tail
                rows_v[r, pl.ds(0, 16)] = raw_v[j, 0, pl.ds(0, 16)]
                rows_v[r, pl.ds(16, 16)] = raw_v[j, 0, pl.ds(16, 16)]
                rows_v[r, pl.ds(32, 16)] = raw_v[j, 0, pl.ds(32, 16)]
                rows_v[r, pl.ds(40, 16)] = raw_v[j, 0, pl.ds(40, 16)]
                return cc

            lax.fori_loop(0, _K, body, 0)
            return carry

        lax.fori_loop(0, n_ch, chunk, 0, unroll=False)

        pltpu.sync_copy(rows_v, out_hbm.at[pl.ds(base, b_per_w)])

    return k


def kernel(x, table, shared):
    table3 = jnp.reshape(table, (table.shape[0] // _BLK, _BLK, _D_TAB))
    tail16 = jnp.tile(jnp.reshape(shared, (_D_OUT - _D_TAB,)), 2)
    out = _make_sc_kernel()(x.astype(jnp.int32), table3, tail16)
    return out[:, None, :]
